# Initial kernel scaffold; baseline (speedup 1.0000x reference)
import jax
import jax.numpy as jnp
from jax.experimental import pallas as pl


def kernel(x, edge_index, W1, b1, W2, b2):
    raise NotImplementedError("write your pallas kernel here")


# speedup vs baseline: 27.5159x; 27.5159x over previous
import functools

import jax
import jax.numpy as jnp
from jax import lax
from jax.experimental import pallas as pl
from jax.experimental.pallas import tpu as pltpu
from jax.experimental.pallas import tpu_sc as plsc

NC = 2
NS = 16
NW = NC * NS

_MESH = plsc.VectorSubcoreMesh(core_axis_name="c", subcore_axis_name="s")


def _make_deg_kernel(n, ch, k):
    npad = ((n + NS * 16 - 1) // (NS * 16)) * (NS * 16)
    rpt = npad // NS

    @functools.partial(
        pl.kernel,
        out_type=jax.ShapeDtypeStruct((NC * npad,), jnp.float32),
        mesh=_MESH,
        scratch_types=[
            pltpu.VMEM((ch, k), jnp.int32),
            pltpu.VMEM((128,), jnp.float32),
            pltpu.VMEM((rpt,), jnp.float32),
            pltpu.VMEM_SHARED((npad,), jnp.float32),
        ],
        compiler_params=pltpu.CompilerParams(use_tc_tiling_on_sc=False),
    )
    def deg_kernel(dst_hbm, out_hbm, dst_v, ones_v, stage_v, acc_sh):
        c = lax.axis_index("c")
        s = lax.axis_index("s")
        wid = c * NS + s

        pltpu.sync_copy(dst_hbm.at[wid], dst_v)

        zeros16 = jnp.zeros((16,), jnp.float32)
        ones16 = jnp.ones((16,), jnp.float32)
        for i in range(128 // 16):
            ones_v[pl.ds(i * 16, 16)] = ones16

        def zero_body(i, carry):
            stage_v[pl.ds(i * 16, 16)] = zeros16
            return carry

        lax.fori_loop(0, rpt // 16, zero_body, 0)
        sbase = pl.multiple_of(s * rpt, 8)
        pltpu.sync_copy(stage_v, acc_sh.at[pl.ds(sbase, rpt)])
        plsc.subcore_barrier()

        def edge_body(j, carry):
            pltpu.sync_copy(ones_v.at[pl.ds(0, k)], acc_sh.at[dst_v.at[j]],
                            add=True)
            return carry

        lax.fori_loop(0, ch, edge_body, 0)
        plsc.subcore_barrier()

        pltpu.sync_copy(acc_sh.at[pl.ds(sbase, rpt)], stage_v)
        obase = pl.multiple_of(wid * rpt, 8)
        pltpu.sync_copy(stage_v, out_hbm.at[pl.ds(obase, rpt)])

    return deg_kernel


def _make_prop_kernel(n, d2, ch, k):
    rpb = (n // (NS * 8)) * 8
    rem = n - rpb * NS
    sr = 208
    nst = rpb // sr
    assert nst * sr == rpb and rem % 8 == 0 and rem <= sr

    @functools.partial(
        pl.kernel,
        out_type=jax.ShapeDtypeStruct((NC, n, d2), jnp.float32),
        mesh=_MESH,
        scratch_types=[
            pltpu.VMEM((ch, k), jnp.int32),
            pltpu.VMEM((ch, k), jnp.int32),
            pltpu.VMEM((k, d2), jnp.float32),
            pltpu.VMEM((k, d2), jnp.float32),
            pltpu.VMEM((sr, d2), jnp.float32),
            pltpu.VMEM_SHARED((n, d2), jnp.float32),
            pltpu.SemaphoreType.DMA,
            pltpu.SemaphoreType.DMA,
        ],
        compiler_params=pltpu.CompilerParams(use_tc_tiling_on_sc=False),
    )
    def prop_kernel(xs_hbm, src_hbm, dst_hbm, out_hbm,
                    src_v, dst_v, rows0, rows1, stage_v, acc_sh, sem0, sem1):
        c = lax.axis_index("c")
        s = lax.axis_index("s")

        pltpu.sync_copy(src_hbm.at[s], src_v)
        pltpu.sync_copy(dst_hbm.at[s], dst_v)

        def tile_chunks(fn):
            for p in range(nst):
                fn(pl.multiple_of(s * rpb + p * sr, 8), sr)
            if rem:
                @pl.when(s == NS - 1)
                def _():
                    fn(NS * rpb, rem)

        def seed(off, sz):
            pltpu.sync_copy(xs_hbm.at[c, pl.ds(off, sz)],
                            stage_v.at[pl.ds(0, sz)])
            pltpu.sync_copy(stage_v.at[pl.ds(0, sz)],
                            acc_sh.at[pl.ds(off, sz)])
        tile_chunks(seed)
        plsc.subcore_barrier()

        rows = [rows0, rows1]
        sems = [sem0, sem1]
        pltpu.async_copy(xs_hbm.at[c].at[src_v.at[0]], rows0, sem0)
        pltpu.async_copy(xs_hbm.at[c].at[src_v.at[1]], rows1, sem1)

        def edge_body(jj, carry):
            for b in range(2):
                j = jj * 2 + b
                pltpu.make_async_copy(xs_hbm.at[c].at[src_v.at[j]], rows[b],
                                      sems[b]).wait()
                pltpu.sync_copy(rows[b], acc_sh.at[dst_v.at[j]], add=True)

                @pl.when(j + 2 < ch)
                def _():
                    pltpu.async_copy(xs_hbm.at[c].at[src_v.at[j + 2]],
                                     rows[b], sems[b])
            return carry

        lax.fori_loop(0, ch // 2, edge_body, 0)
        plsc.subcore_barrier()

        def drain(off, sz):
            pltpu.sync_copy(acc_sh.at[pl.ds(off, sz)],
                            stage_v.at[pl.ds(0, sz)])
            pltpu.sync_copy(stage_v.at[pl.ds(0, sz)],
                            out_hbm.at[c, pl.ds(off, sz)])
        tile_chunks(drain)

    return prop_kernel


def _mm_scale_kernel(x_ref, w_ref, d0_ref, d1_ref, xs_ref, dinv_ref):
    h = jnp.dot(x_ref[...], w_ref[...], preferred_element_type=jnp.float32)
    deg = d0_ref[...] + d1_ref[...] + 1.0
    dinv = lax.rsqrt(deg)
    dinv_ref[...] = dinv
    d2 = h.shape[1] // 2
    xs_ref[0] = h[:, :d2] * dinv
    xs_ref[1] = h[:, d2:] * dinv


def _combine_mm_kernel(a0_ref, a1_ref, dinv_ref, b_ref, w_ref, xs_ref):
    dinv = dinv_ref[...]
    d2 = b_ref.shape[1] // 2
    h0 = jnp.maximum(a0_ref[...] * dinv + b_ref[:, :d2], 0.0)
    h1 = jnp.maximum(a1_ref[...] * dinv + b_ref[:, d2:], 0.0)
    y = (jnp.dot(h0, w_ref[:d2, :], preferred_element_type=jnp.float32)
         + jnp.dot(h1, w_ref[d2:, :], preferred_element_type=jnp.float32))
    y = y * dinv
    q2 = y.shape[1] // 2
    xs_ref[0] = y[:, :q2]
    xs_ref[1] = y[:, q2:]


def _final_kernel(a0_ref, a1_ref, dinv_ref, b_ref, out_ref):
    o = jnp.concatenate([a0_ref[...], a1_ref[...]], axis=1)
    o = o * dinv_ref[...] + b_ref[...]
    m = jnp.max(o, axis=1, keepdims=True)
    ex = jnp.exp(o - m)
    lse = jnp.log(jnp.sum(ex, axis=1, keepdims=True)) + m
    out_ref[...] = o - lse


def kernel(x, edge_index, W1, b1, W2, b2):
    n, din = x.shape
    dh = W1.shape[1]
    dout = W2.shape[1]
    e = edge_index.shape[1]

    k = 125
    ch32 = e // (NW * k)
    ch16 = e // (NS * k)
    assert ch32 * NW * k == e

    src16 = edge_index[0].reshape(NS, ch16, k)
    dst16 = edge_index[1].reshape(NS, ch16, k)
    dst32 = edge_index[1].reshape(NW, ch32, k)

    degp = _make_deg_kernel(n, ch32, k)(dst32)
    npad = degp.shape[0] // NC
    degf = degp.reshape(NC, npad)[:, :n]
    d0 = degf[0].reshape(n, 1)
    d1 = degf[1].reshape(n, 1)

    br = 1000
    grid = n // br
    row = lambda d: pl.BlockSpec((br,) + d, lambda i: (i,) + (0,) * len(d))
    half = lambda d: pl.BlockSpec((NC, br, d), lambda i: (0, i, 0))
    full = lambda shp: pl.BlockSpec(shp, lambda i: (0,) * len(shp))

    xs1, dinv = pl.pallas_call(
        _mm_scale_kernel,
        grid=(grid,),
        in_specs=[row((din,)), full((din, dh)), row((1,)), row((1,))],
        out_specs=[half(dh // 2), row((1,))],
        out_shape=[
            jax.ShapeDtypeStruct((NC, n, dh // 2), jnp.float32),
            jax.ShapeDtypeStruct((n, 1), jnp.float32),
        ],
    )(x, W1, d0, d1)

    acc1 = _make_prop_kernel(n, dh // 2, ch16, k)(xs1, src16, dst16)

    xs2 = pl.pallas_call(
        _combine_mm_kernel,
        grid=(grid,),
        in_specs=[row((dh // 2,)), row((dh // 2,)), row((1,)),
                  full((1, dh)), full((dh, dout))],
        out_specs=half(dout // 2),
        out_shape=jax.ShapeDtypeStruct((NC, n, dout // 2), jnp.float32),
    )(acc1[0], acc1[1], dinv, b1.reshape(1, dh), W2)

    acc2 = _make_prop_kernel(n, dout // 2, ch16, k)(xs2, src16, dst16)

    out = pl.pallas_call(
        _final_kernel,
        grid=(grid,),
        in_specs=[row((dout // 2,)), row((dout // 2,)), row((1,)),
                  full((1, dout))],
        out_specs=row((dout,)),
        out_shape=jax.ShapeDtypeStruct((n, dout), jnp.float32),
    )(acc2[0], acc2[1], dinv, b2.reshape(1, dout))

    return out


# speedup vs baseline: 38.7414x; 1.4080x over previous
import functools

import jax
import jax.numpy as jnp
from jax import lax
from jax.experimental import pallas as pl
from jax.experimental.pallas import tpu as pltpu
from jax.experimental.pallas import tpu_sc as plsc

NC = 2
NS = 16
NW = NC * NS
NB = 4

_MESH = plsc.VectorSubcoreMesh(core_axis_name="c", subcore_axis_name="s")


def _make_deg_kernel(n, ch, k):
    npad = ((n + NS * 16 - 1) // (NS * 16)) * (NS * 16)
    rpt = npad // NS

    @functools.partial(
        pl.kernel,
        out_type=jax.ShapeDtypeStruct((NC * npad,), jnp.float32),
        mesh=_MESH,
        scratch_types=[
            pltpu.VMEM((ch, k), jnp.int32),
            pltpu.VMEM((128,), jnp.float32),
            pltpu.VMEM((rpt,), jnp.float32),
            pltpu.VMEM_SHARED((npad,), jnp.float32),
        ],
        compiler_params=pltpu.CompilerParams(use_tc_tiling_on_sc=False),
    )
    def deg_kernel(dst_hbm, out_hbm, dst_v, ones_v, stage_v, acc_sh):
        c = lax.axis_index("c")
        s = lax.axis_index("s")
        wid = c * NS + s

        pltpu.sync_copy(dst_hbm.at[wid], dst_v)

        zeros16 = jnp.zeros((16,), jnp.float32)
        ones16 = jnp.ones((16,), jnp.float32)
        for i in range(128 // 16):
            ones_v[pl.ds(i * 16, 16)] = ones16

        def zero_body(i, carry):
            stage_v[pl.ds(i * 16, 16)] = zeros16
            return carry

        lax.fori_loop(0, rpt // 16, zero_body, 0)
        sbase = pl.multiple_of(s * rpt, 8)
        pltpu.sync_copy(stage_v, acc_sh.at[pl.ds(sbase, rpt)])
        plsc.subcore_barrier()

        def edge_body(j, carry):
            pltpu.sync_copy(ones_v.at[pl.ds(0, k)], acc_sh.at[dst_v.at[j]],
                            add=True)
            return carry

        lax.fori_loop(0, ch, edge_body, 0)
        plsc.subcore_barrier()

        pltpu.sync_copy(acc_sh.at[pl.ds(sbase, rpt)], stage_v)
        obase = pl.multiple_of(wid * rpt, 8)
        pltpu.sync_copy(stage_v, out_hbm.at[pl.ds(obase, rpt)])

    return deg_kernel


def _make_prop_kernel(n, d2, ch, k):
    rpb = (n // (NS * 8)) * 8
    rem = n - rpb * NS
    sr = 208
    nst = rpb // sr
    assert nst * sr == rpb and rem % 8 == 0 and rem <= sr
    assert ch % NB == 0

    @functools.partial(
        pl.kernel,
        out_type=jax.ShapeDtypeStruct((n, 2 * d2), jnp.float32),
        mesh=_MESH,
        scratch_types=[
            pltpu.VMEM((ch, k), jnp.int32),
            pltpu.VMEM((ch, k), jnp.int32),
            [pltpu.VMEM((k, d2), jnp.float32) for _ in range(NB)],
            pltpu.VMEM((sr, d2), jnp.float32),
            pltpu.VMEM_SHARED((n, d2), jnp.float32),
            [pltpu.SemaphoreType.DMA for _ in range(NB)],
            [pltpu.SemaphoreType.DMA for _ in range(NB)],
        ],
        compiler_params=pltpu.CompilerParams(use_tc_tiling_on_sc=False),
    )
    def prop_kernel(t2_hbm, idx_hbm, dst_hbm, out_hbm,
                    idx_v, dst_v, rows, stage_v, acc_sh, gsems, ssems):
        c = lax.axis_index("c")
        s = lax.axis_index("s")

        pltpu.sync_copy(idx_hbm.at[c, s], idx_v)
        pltpu.sync_copy(dst_hbm.at[s], dst_v)

        def tile_chunks(fn):
            for p in range(nst):
                fn(pl.multiple_of(s * rpb + p * sr, 8), sr)
            if rem:
                @pl.when(s == NS - 1)
                def _():
                    fn(NS * rpb, rem)

        zeros16 = jnp.zeros((16,), jnp.float32)

        def zero_body(i, carry):
            for t in range(d2 // 16):
                stage_v[i, pl.ds(t * 16, 16)] = zeros16
            return carry

        lax.fori_loop(0, sr, zero_body, 0)

        def seed0(off, sz):
            pltpu.sync_copy(stage_v.at[pl.ds(0, sz)],
                            acc_sh.at[pl.ds(off, sz)])
        tile_chunks(seed0)
        plsc.subcore_barrier()

        def issue_gather(j, b):
            pltpu.async_copy(t2_hbm.at[idx_v.at[j]], rows[b], gsems[b])

        def wait_gather(b):
            pltpu.make_async_copy(t2_hbm.at[idx_v.at[0]], rows[b],
                                  gsems[b]).wait()

        def issue_scatter(j, b):
            pltpu.async_copy(rows[b], acc_sh.at[dst_v.at[j]], ssems[b],
                             add=True)

        def wait_scatter(b):
            pltpu.make_async_copy(rows[b], acc_sh.at[dst_v.at[0]],
                                  ssems[b]).wait()

        issue_gather(0, 0)
        issue_gather(1, 1)

        def edge_body(jj, carry):
            for b in range(NB):
                j = jj * NB + b
                bp = (b + 2) % NB

                @pl.when(j >= 2)
                def _():
                    wait_scatter(bp)

                @pl.when(j + 2 < ch)
                def _():
                    issue_gather(j + 2, bp)

                wait_gather(b)
                issue_scatter(j, b)
            return carry

        lax.fori_loop(0, ch // NB, edge_body, 0)
        wait_scatter((ch - 2) % NB)
        wait_scatter((ch - 1) % NB)
        plsc.subcore_barrier()

        def drain(off, sz):
            pltpu.sync_copy(acc_sh.at[pl.ds(off, sz)],
                            stage_v.at[pl.ds(0, sz)])
            pltpu.sync_copy(stage_v.at[pl.ds(0, sz)],
                            out_hbm.at[pl.ds(off, sz), pl.ds(c * d2, d2)])
        tile_chunks(drain)

    return prop_kernel


def _mm_scale_kernel(x_ref, w_ref, d0_ref, d1_ref, xs_ref, dinv_ref):
    h = jnp.dot(x_ref[...], w_ref[...], preferred_element_type=jnp.float32)
    deg = d0_ref[...] + d1_ref[...] + 1.0
    dinv = lax.rsqrt(deg)
    dinv_ref[...] = dinv
    xs_ref[...] = h * dinv


def _combine_mm_kernel(a_ref, sl_ref, dinv_ref, b_ref, w_ref, xs_ref):
    dinv = dinv_ref[...]
    h = jnp.maximum((a_ref[...] + sl_ref[...]) * dinv + b_ref[...], 0.0)
    y = jnp.dot(h, w_ref[...], preferred_element_type=jnp.float32) * dinv
    xs_ref[...] = y


def _final_kernel(a_ref, sl_ref, dinv_ref, b_ref, out_ref):
    o = a_ref[...] + sl_ref[...]
    o = o * dinv_ref[...] + b_ref[...]
    m = jnp.max(o, axis=1, keepdims=True)
    ex = jnp.exp(o - m)
    lse = jnp.log(jnp.sum(ex, axis=1, keepdims=True)) + m
    out_ref[...] = o - lse


def kernel(x, edge_index, W1, b1, W2, b2):
    n, din = x.shape
    dh = W1.shape[1]
    dout = W2.shape[1]
    e = edge_index.shape[1]

    k = 125
    ch32 = e // (NW * k)
    ch16 = e // (NS * k)
    assert ch32 * NW * k == e

    src = edge_index[0]
    idx2 = jnp.stack([src * 2, src * 2 + 1]).reshape(NC, NS, ch16, k)
    dst16 = edge_index[1].reshape(NS, ch16, k)
    dst32 = edge_index[1].reshape(NW, ch32, k)

    degp = _make_deg_kernel(n, ch32, k)(dst32)
    npad = degp.shape[0] // NC
    degf = degp.reshape(NC, npad)[:, :n]
    d0 = degf[0].reshape(n, 1)
    d1 = degf[1].reshape(n, 1)

    br = 2000
    grid = n // br
    row = lambda d: pl.BlockSpec((br,) + d, lambda i: (i,) + (0,) * len(d))
    half = lambda d: pl.BlockSpec((br // 2, d), lambda i: (i, 0))
    full = lambda shp: pl.BlockSpec(shp, lambda i: (0,) * len(shp))

    xs1, dinv = pl.pallas_call(
        _mm_scale_kernel,
        grid=(grid,),
        in_specs=[row((din,)), full((din, dh)), row((1,)), row((1,))],
        out_specs=[row((dh,)), row((1,))],
        out_shape=[
            jax.ShapeDtypeStruct((n, dh), jnp.float32),
            jax.ShapeDtypeStruct((n, 1), jnp.float32),
        ],
    )(x, W1, d0, d1)

    prop1 = _make_prop_kernel(n, dh // 2, ch16, k)
    osum1 = prop1(xs1.reshape(2 * n, dh // 2), idx2, dst16)

    xs2 = pl.pallas_call(
        _combine_mm_kernel,
        grid=(grid,),
        in_specs=[row((dh,)), row((dh,)), row((1,)), full((1, dh)),
                  full((dh, dout))],
        out_specs=row((dout,)),
        out_shape=jax.ShapeDtypeStruct((n, dout), jnp.float32),
    )(osum1, xs1, dinv, b1.reshape(1, dh), W2)

    prop2 = _make_prop_kernel(n, dout // 2, ch16, k)
    osum2 = prop2(xs2.reshape(2 * n, dout // 2), idx2, dst16)

    out = pl.pallas_call(
        _final_kernel,
        grid=(grid,),
        in_specs=[row((dout,)), row((dout,)), row((1,)), full((1, dout))],
        out_specs=row((dout,)),
        out_shape=jax.ShapeDtypeStruct((n, dout), jnp.float32),
    )(osum2, xs2, dinv, b2.reshape(1, dout))

    return out


# speedup vs baseline: 39.9015x; 1.0299x over previous
import functools

import jax
import jax.numpy as jnp
from jax import lax
from jax.experimental import pallas as pl
from jax.experimental.pallas import tpu as pltpu
from jax.experimental.pallas import tpu_sc as plsc

NC = 2
NS = 16
NW = NC * NS

_MESH = plsc.VectorSubcoreMesh(core_axis_name="c", subcore_axis_name="s")


def _make_deg_kernel(n, ch, k):
    npad = ((n + NS * 16 - 1) // (NS * 16)) * (NS * 16)
    rpt = npad // NS
    chh = ch // 2

    @functools.partial(
        pl.kernel,
        out_type=jax.ShapeDtypeStruct((NC * npad,), jnp.float32),
        mesh=_MESH,
        scratch_types=[
            pltpu.VMEM((chh, k), jnp.int32),
            pltpu.VMEM((128,), jnp.float32),
            pltpu.VMEM((rpt,), jnp.float32),
            pltpu.VMEM_SHARED((npad,), jnp.float32),
        ],
        compiler_params=pltpu.CompilerParams(use_tc_tiling_on_sc=False),
    )
    def deg_kernel(dst_hbm, out_hbm, dst_v, ones_v, stage_v, acc_sh):
        c = lax.axis_index("c")
        s = lax.axis_index("s")
        wid = c * NS + s

        pltpu.sync_copy(dst_hbm.at[s, pl.ds(c * chh, chh)], dst_v)

        zeros16 = jnp.zeros((16,), jnp.float32)
        ones16 = jnp.ones((16,), jnp.float32)
        for i in range(128 // 16):
            ones_v[pl.ds(i * 16, 16)] = ones16

        def zero_body(i, carry):
            stage_v[pl.ds(i * 16, 16)] = zeros16
            return carry

        lax.fori_loop(0, rpt // 16, zero_body, 0)
        sbase = pl.multiple_of(s * rpt, 8)
        pltpu.sync_copy(stage_v, acc_sh.at[pl.ds(sbase, rpt)])
        plsc.subcore_barrier()

        def edge_body(j, carry):
            pltpu.sync_copy(ones_v.at[pl.ds(0, k)], acc_sh.at[dst_v.at[j]],
                            add=True)
            return carry

        lax.fori_loop(0, chh, edge_body, 0)
        plsc.subcore_barrier()

        pltpu.sync_copy(acc_sh.at[pl.ds(sbase, rpt)], stage_v)
        obase = pl.multiple_of(wid * rpt, 8)
        pltpu.sync_copy(stage_v, out_hbm.at[pl.ds(obase, rpt)])

    return deg_kernel


def _make_prop_kernel(n, d2, ch, k, NB):
    NL = NB // 2
    rpb = (n // (NS * 8)) * 8
    rem = n - rpb * NS
    sr = 208
    nst = rpb // sr
    assert nst * sr == rpb and rem % 8 == 0 and rem <= sr
    assert ch % NB == 0

    @functools.partial(
        pl.kernel,
        out_type=jax.ShapeDtypeStruct((n, 2 * d2), jnp.float32),
        mesh=_MESH,
        scratch_types=[
            pltpu.VMEM((ch, k), jnp.int32),
            pltpu.VMEM((ch, k), jnp.int32),
            [pltpu.VMEM((k, d2), jnp.float32) for _ in range(NB)],
            pltpu.VMEM((sr, d2), jnp.float32),
            pltpu.VMEM_SHARED((n, d2), jnp.float32),
            [pltpu.SemaphoreType.DMA for _ in range(NB)],
            [pltpu.SemaphoreType.DMA for _ in range(NB)],
        ],
        compiler_params=pltpu.CompilerParams(use_tc_tiling_on_sc=False),
    )
    def prop_kernel(t2_hbm, idx_hbm, dst_hbm, out_hbm,
                    idx_v, dst_v, rows, stage_v, acc_sh, gsems, ssems):
        c = lax.axis_index("c")
        s = lax.axis_index("s")

        pltpu.sync_copy(idx_hbm.at[c, s], idx_v)
        pltpu.sync_copy(dst_hbm.at[s], dst_v)

        def tile_chunks(fn):
            for p in range(nst):
                fn(pl.multiple_of(s * rpb + p * sr, 8), sr)
            if rem:
                @pl.when(s == NS - 1)
                def _():
                    fn(NS * rpb, rem)

        zeros16 = jnp.zeros((16,), jnp.float32)

        def zero_body(i, carry):
            for t in range(d2 // 16):
                stage_v[i, pl.ds(t * 16, 16)] = zeros16
            return carry

        lax.fori_loop(0, sr, zero_body, 0)

        def seed0(off, sz):
            pltpu.sync_copy(stage_v.at[pl.ds(0, sz)],
                            acc_sh.at[pl.ds(off, sz)])
        tile_chunks(seed0)
        plsc.subcore_barrier()

        def issue_gather(j, b):
            pltpu.async_copy(t2_hbm.at[idx_v.at[j]], rows[b], gsems[b])

        def wait_gather(b):
            pltpu.make_async_copy(t2_hbm.at[idx_v.at[0]], rows[b],
                                  gsems[b]).wait()

        def issue_scatter(j, b):
            pltpu.async_copy(rows[b], acc_sh.at[dst_v.at[j]], ssems[b],
                             add=True)

        def wait_scatter(b):
            pltpu.make_async_copy(rows[b], acc_sh.at[dst_v.at[0]],
                                  ssems[b]).wait()

        for b0 in range(NL):
            issue_gather(b0, b0)

        def edge_body(jj, carry):
            for b in range(NB):
                j = jj * NB + b
                bp = (b + NL) % NB

                @pl.when(j >= NL)
                def _():
                    wait_scatter(bp)

                @pl.when(j + NL < ch)
                def _():
                    issue_gather(j + NL, bp)

                wait_gather(b)
                issue_scatter(j, b)
            return carry

        lax.fori_loop(0, ch // NB, edge_body, 0)
        for j0 in range(ch - NL, ch):
            wait_scatter(j0 % NB)
        plsc.subcore_barrier()

        def drain(off, sz):
            pltpu.sync_copy(acc_sh.at[pl.ds(off, sz)],
                            stage_v.at[pl.ds(0, sz)])
            pltpu.sync_copy(stage_v.at[pl.ds(0, sz)],
                            out_hbm.at[pl.ds(off, sz), pl.ds(c * d2, d2)])
        tile_chunks(drain)

    return prop_kernel


def _mm_scale_kernel(x_ref, w_ref, d0_ref, d1_ref, xs_ref, dinv_ref):
    h = jnp.dot(x_ref[...], w_ref[...], preferred_element_type=jnp.float32)
    deg = d0_ref[...] + d1_ref[...] + 1.0
    dinv = lax.rsqrt(deg)
    dinv_ref[...] = dinv
    xs_ref[...] = h * dinv


def _combine_mm_kernel(a_ref, sl_ref, dinv_ref, b_ref, w_ref, xs_ref):
    dinv = dinv_ref[...]
    h = jnp.maximum((a_ref[...] + sl_ref[...]) * dinv + b_ref[...], 0.0)
    y = jnp.dot(h, w_ref[...], preferred_element_type=jnp.float32) * dinv
    xs_ref[...] = y


def _final_kernel(a_ref, sl_ref, dinv_ref, b_ref, out_ref):
    o = a_ref[...] + sl_ref[...]
    o = o * dinv_ref[...] + b_ref[...]
    m = jnp.max(o, axis=1, keepdims=True)
    ex = jnp.exp(o - m)
    lse = jnp.log(jnp.sum(ex, axis=1, keepdims=True)) + m
    out_ref[...] = o - lse


def kernel(x, edge_index, W1, b1, W2, b2):
    n, din = x.shape
    dh = W1.shape[1]
    dout = W2.shape[1]
    e = edge_index.shape[1]

    k = 125
    ch32 = e // (NW * k)
    ch16 = e // (NS * k)
    assert ch32 * NW * k == e

    src = edge_index[0]
    idx2 = jnp.stack([src * 2, src * 2 + 1]).reshape(NC, NS, ch16, k)
    dst16 = edge_index[1].reshape(NS, ch16, k)

    degp = _make_deg_kernel(n, ch16, k)(dst16)
    npad = degp.shape[0] // NC
    degf = degp.reshape(NC, npad)[:, :n]
    d0 = degf[0].reshape(n, 1)
    d1 = degf[1].reshape(n, 1)

    br = 2000
    grid = n // br
    row = lambda d: pl.BlockSpec((br,) + d, lambda i: (i,) + (0,) * len(d))
    half = lambda d: pl.BlockSpec((br // 2, d), lambda i: (i, 0))
    full = lambda shp: pl.BlockSpec(shp, lambda i: (0,) * len(shp))

    xs1, dinv = pl.pallas_call(
        _mm_scale_kernel,
        grid=(grid,),
        in_specs=[row((din,)), full((din, dh)), row((1,)), row((1,))],
        out_specs=[row((dh,)), row((1,))],
        out_shape=[
            jax.ShapeDtypeStruct((n, dh), jnp.float32),
            jax.ShapeDtypeStruct((n, 1), jnp.float32),
        ],
    )(x, W1, d0, d1)

    prop1 = _make_prop_kernel(n, dh // 2, ch16, k, 4)
    osum1 = prop1(xs1.reshape(2 * n, dh // 2), idx2, dst16)

    xs2 = pl.pallas_call(
        _combine_mm_kernel,
        grid=(grid,),
        in_specs=[row((dh,)), row((dh,)), row((1,)), full((1, dh)),
                  full((dh, dout))],
        out_specs=row((dout,)),
        out_shape=jax.ShapeDtypeStruct((n, dout), jnp.float32),
    )(osum1, xs1, dinv, b1.reshape(1, dh), W2)

    prop2 = _make_prop_kernel(n, dout // 2, ch16, k, 8)
    osum2 = prop2(xs2.reshape(2 * n, dout // 2), idx2, dst16)

    out = pl.pallas_call(
        _final_kernel,
        grid=(grid,),
        in_specs=[row((dout,)), row((dout,)), row((1,)), full((1, dout))],
        out_specs=row((dout,)),
        out_shape=jax.ShapeDtypeStruct((n, dout), jnp.float32),
    )(osum2, xs2, dinv, b2.reshape(1, dout))

    return out


# speedup vs baseline: 40.4689x; 1.0142x over previous
import functools

import jax
import jax.numpy as jnp
from jax import lax
from jax.experimental import pallas as pl
from jax.experimental.pallas import tpu as pltpu
from jax.experimental.pallas import tpu_sc as plsc

NC = 2
NS = 16
NW = NC * NS

_MESH = plsc.VectorSubcoreMesh(core_axis_name="c", subcore_axis_name="s")


def _make_deg_kernel(n, ch, k):
    npad = ((n + NS * 16 - 1) // (NS * 16)) * (NS * 16)
    rpt = npad // NS
    chh = ch // 2

    @functools.partial(
        pl.kernel,
        out_type=jax.ShapeDtypeStruct((NC * npad,), jnp.float32),
        mesh=_MESH,
        scratch_types=[
            pltpu.VMEM((chh, k), jnp.int32),
            pltpu.VMEM((128,), jnp.float32),
            pltpu.VMEM((rpt,), jnp.float32),
            pltpu.VMEM_SHARED((npad,), jnp.float32),
            pltpu.SemaphoreType.DMA,
        ],
        compiler_params=pltpu.CompilerParams(use_tc_tiling_on_sc=False),
    )
    def deg_kernel(dst_hbm, out_hbm, dst_v, ones_v, stage_v, acc_sh, dsem):
        c = lax.axis_index("c")
        s = lax.axis_index("s")
        wid = c * NS + s

        pltpu.sync_copy(dst_hbm.at[s, pl.ds(c * chh, chh)], dst_v)

        zeros16 = jnp.zeros((16,), jnp.float32)
        ones16 = jnp.ones((16,), jnp.float32)
        for i in range(128 // 16):
            ones_v[pl.ds(i * 16, 16)] = ones16

        def zero_body(i, carry):
            stage_v[pl.ds(i * 16, 16)] = zeros16
            return carry

        lax.fori_loop(0, rpt // 16, zero_body, 0)
        sbase = pl.multiple_of(s * rpt, 8)
        pltpu.sync_copy(stage_v, acc_sh.at[pl.ds(sbase, rpt)])
        plsc.subcore_barrier()

        def edge_body(j, carry):
            pltpu.async_copy(ones_v.at[pl.ds(0, k)], acc_sh.at[dst_v.at[j]],
                             dsem, add=True)
            return carry

        lax.fori_loop(0, chh, edge_body, 0)

        def edge_drain(j, carry):
            pltpu.make_async_copy(ones_v.at[pl.ds(0, k)],
                                  acc_sh.at[dst_v.at[0]], dsem).wait()
            return carry

        lax.fori_loop(0, chh, edge_drain, 0)
        plsc.subcore_barrier()

        pltpu.sync_copy(acc_sh.at[pl.ds(sbase, rpt)], stage_v)
        obase = pl.multiple_of(wid * rpt, 8)
        pltpu.sync_copy(stage_v, out_hbm.at[pl.ds(obase, rpt)])

    return deg_kernel


def _make_prop_kernel(n, d2, ch, k, NB):
    NL = NB // 2
    rpb = (n // (NS * 8)) * 8
    rem = n - rpb * NS
    sr = 208
    nst = rpb // sr
    assert nst * sr == rpb and rem % 8 == 0 and rem <= sr
    assert ch % NB == 0

    @functools.partial(
        pl.kernel,
        out_type=jax.ShapeDtypeStruct((n, 2 * d2), jnp.float32),
        mesh=_MESH,
        scratch_types=[
            pltpu.VMEM((ch, k), jnp.int32),
            pltpu.VMEM((ch, k), jnp.int32),
            [pltpu.VMEM((k, d2), jnp.float32) for _ in range(NB)],
            pltpu.VMEM((sr, d2), jnp.float32),
            pltpu.VMEM_SHARED((n, d2), jnp.float32),
            [pltpu.SemaphoreType.DMA for _ in range(NB)],
            [pltpu.SemaphoreType.DMA for _ in range(NB)],
        ],
        compiler_params=pltpu.CompilerParams(use_tc_tiling_on_sc=False),
    )
    def prop_kernel(t2_hbm, idx_hbm, dst_hbm, out_hbm,
                    idx_v, dst_v, rows, stage_v, acc_sh, gsems, ssems):
        c = lax.axis_index("c")
        s = lax.axis_index("s")

        pltpu.sync_copy(idx_hbm.at[c, s], idx_v)
        pltpu.sync_copy(dst_hbm.at[s], dst_v)

        def tile_chunks(fn):
            for p in range(nst):
                fn(pl.multiple_of(s * rpb + p * sr, 8), sr)
            if rem:
                @pl.when(s == NS - 1)
                def _():
                    fn(NS * rpb, rem)

        zeros16 = jnp.zeros((16,), jnp.float32)

        def zero_body(i, carry):
            for t in range(d2 // 16):
                stage_v[i, pl.ds(t * 16, 16)] = zeros16
            return carry

        lax.fori_loop(0, sr, zero_body, 0)

        def seed0(off, sz):
            pltpu.sync_copy(stage_v.at[pl.ds(0, sz)],
                            acc_sh.at[pl.ds(off, sz)])
        tile_chunks(seed0)
        plsc.subcore_barrier()

        def issue_gather(j, b):
            pltpu.async_copy(t2_hbm.at[idx_v.at[j]], rows[b], gsems[b])

        def wait_gather(b):
            pltpu.make_async_copy(t2_hbm.at[idx_v.at[0]], rows[b],
                                  gsems[b]).wait()

        def issue_scatter(j, b):
            pltpu.async_copy(rows[b], acc_sh.at[dst_v.at[j]], ssems[b],
                             add=True)

        def wait_scatter(b):
            pltpu.make_async_copy(rows[b], acc_sh.at[dst_v.at[0]],
                                  ssems[b]).wait()

        for b0 in range(NL):
            issue_gather(b0, b0)

        def edge_body(jj, carry):
            for b in range(NB):
                j = jj * NB + b
                bp = (b + NL) % NB

                @pl.when(j >= NL)
                def _():
                    wait_scatter(bp)

                @pl.when(j + NL < ch)
                def _():
                    issue_gather(j + NL, bp)

                wait_gather(b)
                issue_scatter(j, b)
            return carry

        lax.fori_loop(0, ch // NB, edge_body, 0)
        for j0 in range(ch - NL, ch):
            wait_scatter(j0 % NB)
        plsc.subcore_barrier()

        def drain(off, sz):
            pltpu.sync_copy(acc_sh.at[pl.ds(off, sz)],
                            stage_v.at[pl.ds(0, sz)])
            pltpu.sync_copy(stage_v.at[pl.ds(0, sz)],
                            out_hbm.at[pl.ds(off, sz), pl.ds(c * d2, d2)])
        tile_chunks(drain)

    return prop_kernel


def _mm_scale_kernel(x_ref, w_ref, d0_ref, d1_ref, xs_ref, dinv_ref):
    h = jnp.dot(x_ref[...], w_ref[...], preferred_element_type=jnp.float32)
    deg = d0_ref[...] + d1_ref[...] + 1.0
    dinv = lax.rsqrt(deg)
    dinv_ref[...] = dinv
    xs_ref[...] = h * dinv


def _combine_mm_kernel(a_ref, sl_ref, dinv_ref, b_ref, w_ref, xs_ref):
    dinv = dinv_ref[...]
    h = jnp.maximum((a_ref[...] + sl_ref[...]) * dinv + b_ref[...], 0.0)
    y = jnp.dot(h, w_ref[...], preferred_element_type=jnp.float32) * dinv
    xs_ref[...] = y


def _final_kernel(a_ref, sl_ref, dinv_ref, b_ref, out_ref):
    o = a_ref[...] + sl_ref[...]
    o = o * dinv_ref[...] + b_ref[...]
    m = jnp.max(o, axis=1, keepdims=True)
    ex = jnp.exp(o - m)
    lse = jnp.log(jnp.sum(ex, axis=1, keepdims=True)) + m
    out_ref[...] = o - lse


def kernel(x, edge_index, W1, b1, W2, b2):
    n, din = x.shape
    dh = W1.shape[1]
    dout = W2.shape[1]
    e = edge_index.shape[1]

    k = 125
    ch32 = e // (NW * k)
    ch16 = e // (NS * k)
    assert ch32 * NW * k == e

    src = edge_index[0]
    (idx2,) = lax.optimization_barrier(
        (jnp.stack([src * 2, src * 2 + 1]).reshape(NC, NS, ch16, k),))
    dst16 = edge_index[1].reshape(NS, ch16, k)

    degp = _make_deg_kernel(n, ch16, k)(dst16)
    npad = degp.shape[0] // NC
    degf = degp.reshape(NC, npad)[:, :n]
    d0 = degf[0].reshape(n, 1)
    d1 = degf[1].reshape(n, 1)

    br = 2000
    grid = n // br
    row = lambda d: pl.BlockSpec((br,) + d, lambda i: (i,) + (0,) * len(d))
    half = lambda d: pl.BlockSpec((br // 2, d), lambda i: (i, 0))
    full = lambda shp: pl.BlockSpec(shp, lambda i: (0,) * len(shp))

    xs1, dinv = pl.pallas_call(
        _mm_scale_kernel,
        grid=(grid,),
        in_specs=[row((din,)), full((din, dh)), row((1,)), row((1,))],
        out_specs=[row((dh,)), row((1,))],
        out_shape=[
            jax.ShapeDtypeStruct((n, dh), jnp.float32),
            jax.ShapeDtypeStruct((n, 1), jnp.float32),
        ],
    )(x, W1, d0, d1)

    prop1 = _make_prop_kernel(n, dh // 2, ch16, k, 4)
    osum1 = prop1(xs1.reshape(2 * n, dh // 2), idx2, dst16)

    xs2 = pl.pallas_call(
        _combine_mm_kernel,
        grid=(grid,),
        in_specs=[row((dh,)), row((dh,)), row((1,)), full((1, dh)),
                  full((dh, dout))],
        out_specs=row((dout,)),
        out_shape=jax.ShapeDtypeStruct((n, dout), jnp.float32),
    )(osum1, xs1, dinv, b1.reshape(1, dh), W2)

    prop2 = _make_prop_kernel(n, dout // 2, ch16, k, 8)
    osum2 = prop2(xs2.reshape(2 * n, dout // 2), idx2, dst16)

    out = pl.pallas_call(
        _final_kernel,
        grid=(grid,),
        in_specs=[row((dout,)), row((dout,)), row((1,)), full((1, dout))],
        out_specs=row((dout,)),
        out_shape=jax.ShapeDtypeStruct((n, dout), jnp.float32),
    )(osum2, xs2, dinv, b2.reshape(1, dout))

    return out


# speedup vs baseline: 40.5450x; 1.0019x over previous
import functools

import jax
import jax.numpy as jnp
from jax import lax
from jax.experimental import pallas as pl
from jax.experimental.pallas import tpu as pltpu
from jax.experimental.pallas import tpu_sc as plsc

NC = 2
NS = 16
NW = NC * NS

_MESH = plsc.VectorSubcoreMesh(core_axis_name="c", subcore_axis_name="s")


def _make_deg_kernel(n, ch, k):
    npad = ((n + NS * 16 - 1) // (NS * 16)) * (NS * 16)
    rpt = npad // NS
    chh = ch // 2

    @functools.partial(
        pl.kernel,
        out_type=jax.ShapeDtypeStruct((NC * npad,), jnp.float32),
        mesh=_MESH,
        scratch_types=[
            pltpu.VMEM((chh, k), jnp.int32),
            pltpu.VMEM((128,), jnp.float32),
            pltpu.VMEM((rpt,), jnp.float32),
            pltpu.VMEM_SHARED((npad,), jnp.float32),
            pltpu.SemaphoreType.DMA,
        ],
        compiler_params=pltpu.CompilerParams(use_tc_tiling_on_sc=False),
    )
    def deg_kernel(dst_hbm, out_hbm, dst_v, ones_v, stage_v, acc_sh, dsem):
        c = lax.axis_index("c")
        s = lax.axis_index("s")
        wid = c * NS + s

        pltpu.sync_copy(dst_hbm.at[s, pl.ds(c * chh, chh)], dst_v)

        zeros16 = jnp.zeros((16,), jnp.float32)
        ones16 = jnp.ones((16,), jnp.float32)
        for i in range(128 // 16):
            ones_v[pl.ds(i * 16, 16)] = ones16

        def zero_body(i, carry):
            stage_v[pl.ds(i * 16, 16)] = zeros16
            return carry

        lax.fori_loop(0, rpt // 16, zero_body, 0)
        sbase = pl.multiple_of(s * rpt, 8)
        pltpu.sync_copy(stage_v, acc_sh.at[pl.ds(sbase, rpt)])
        plsc.subcore_barrier()

        def edge_body(j, carry):
            pltpu.async_copy(ones_v.at[pl.ds(0, k)], acc_sh.at[dst_v.at[j]],
                             dsem, add=True)
            return carry

        lax.fori_loop(0, chh, edge_body, 0)

        def edge_drain(j, carry):
            pltpu.make_async_copy(ones_v.at[pl.ds(0, k)],
                                  acc_sh.at[dst_v.at[0]], dsem).wait()
            return carry

        lax.fori_loop(0, chh, edge_drain, 0)
        plsc.subcore_barrier()

        pltpu.sync_copy(acc_sh.at[pl.ds(sbase, rpt)], stage_v)
        obase = pl.multiple_of(wid * rpt, 8)
        pltpu.sync_copy(stage_v, out_hbm.at[pl.ds(obase, rpt)])

    return deg_kernel


def _make_prop_kernel(n, d2, ch, k, NB):
    NL = NB // 2
    rpb = (n // (NS * 8)) * 8
    rem = n - rpb * NS
    sr = 208
    nst = rpb // sr
    assert nst * sr == rpb and rem % 8 == 0 and rem <= sr
    assert ch % NB == 0

    @functools.partial(
        pl.kernel,
        out_type=jax.ShapeDtypeStruct((n, 2 * d2), jnp.float32),
        mesh=_MESH,
        scratch_types=[
            pltpu.VMEM((ch, k), jnp.int32),
            pltpu.VMEM((ch, k), jnp.int32),
            [pltpu.VMEM((k, d2), jnp.float32) for _ in range(NB)],
            pltpu.VMEM((sr, d2), jnp.float32),
            pltpu.VMEM_SHARED((n, d2), jnp.float32),
            [pltpu.SemaphoreType.DMA for _ in range(NB)],
            [pltpu.SemaphoreType.DMA for _ in range(NB)],
        ],
        compiler_params=pltpu.CompilerParams(use_tc_tiling_on_sc=False),
    )
    def prop_kernel(t2_hbm, idx_hbm, dst_hbm, out_hbm,
                    idx_v, dst_v, rows, stage_v, acc_sh, gsems, ssems):
        c = lax.axis_index("c")
        s = lax.axis_index("s")

        pltpu.sync_copy(idx_hbm.at[c, s], idx_v)
        pltpu.sync_copy(dst_hbm.at[s], dst_v)

        def tile_chunks(fn):
            for p in range(nst):
                fn(pl.multiple_of(s * rpb + p * sr, 8), sr)
            if rem:
                @pl.when(s == NS - 1)
                def _():
                    fn(NS * rpb, rem)

        zeros16 = jnp.zeros((16,), jnp.float32)

        def zero_body(i, carry):
            for t in range(d2 // 16):
                stage_v[i, pl.ds(t * 16, 16)] = zeros16
            return carry

        lax.fori_loop(0, sr, zero_body, 0)

        def seed0(off, sz):
            pltpu.sync_copy(stage_v.at[pl.ds(0, sz)],
                            acc_sh.at[pl.ds(off, sz)])
        tile_chunks(seed0)
        plsc.subcore_barrier()

        def issue_gather(j, b):
            pltpu.async_copy(t2_hbm.at[idx_v.at[j]], rows[b], gsems[b])

        def wait_gather(b):
            pltpu.make_async_copy(t2_hbm.at[idx_v.at[0]], rows[b],
                                  gsems[b]).wait()

        def issue_scatter(j, b):
            pltpu.async_copy(rows[b], acc_sh.at[dst_v.at[j]], ssems[b],
                             add=True)

        def wait_scatter(b):
            pltpu.make_async_copy(rows[b], acc_sh.at[dst_v.at[0]],
                                  ssems[b]).wait()

        for b0 in range(NL):
            issue_gather(b0, b0)

        def edge_body(jj, carry):
            for b in range(NB):
                j = jj * NB + b
                bp = (b + NL) % NB

                @pl.when(j >= NL)
                def _():
                    wait_scatter(bp)

                @pl.when(j + NL < ch)
                def _():
                    issue_gather(j + NL, bp)

                wait_gather(b)
                issue_scatter(j, b)
            return carry

        lax.fori_loop(0, ch // NB, edge_body, 0)
        for j0 in range(ch - NL, ch):
            wait_scatter(j0 % NB)
        plsc.subcore_barrier()

        def drain(off, sz):
            pltpu.sync_copy(acc_sh.at[pl.ds(off, sz)],
                            stage_v.at[pl.ds(0, sz)])
            pltpu.sync_copy(stage_v.at[pl.ds(0, sz)],
                            out_hbm.at[pl.ds(off, sz), pl.ds(c * d2, d2)])
        tile_chunks(drain)

    return prop_kernel


def _mm_scale_kernel(x_ref, w_ref, d0_ref, d1_ref, xs_ref, dinv_ref):
    h = jnp.dot(x_ref[...], w_ref[...], preferred_element_type=jnp.float32)
    deg = d0_ref[...] + d1_ref[...] + 1.0
    dinv = lax.rsqrt(deg)
    dinv_ref[...] = dinv
    xs_ref[...] = h * dinv


def _combine_mm_kernel(a_ref, sl_ref, dinv_ref, b_ref, w_ref, xs_ref):
    dinv = dinv_ref[...]
    h = jnp.maximum((a_ref[...] + sl_ref[...]) * dinv + b_ref[...], 0.0)
    y = jnp.dot(h, w_ref[...], preferred_element_type=jnp.float32) * dinv
    xs_ref[...] = y


def _final_kernel(a_ref, sl_ref, dinv_ref, b_ref, out_ref):
    o = a_ref[...] + sl_ref[...]
    o = o * dinv_ref[...] + b_ref[...]
    m = jnp.max(o, axis=1, keepdims=True)
    ex = jnp.exp(o - m)
    lse = jnp.log(jnp.sum(ex, axis=1, keepdims=True)) + m
    out_ref[...] = o - lse


def kernel(x, edge_index, W1, b1, W2, b2):
    n, din = x.shape
    dh = W1.shape[1]
    dout = W2.shape[1]
    e = edge_index.shape[1]

    k = 125
    ch32 = e // (NW * k)
    ch16 = e // (NS * k)
    assert ch32 * NW * k == e

    src = edge_index[0]
    idx2 = jnp.stack([src * 2, src * 2 + 1]).reshape(NC, NS, ch16, k)
    dst16 = edge_index[1].reshape(NS, ch16, k)

    degp = _make_deg_kernel(n, ch16, k)(dst16)
    npad = degp.shape[0] // NC
    degf = degp.reshape(NC, npad)[:, :n]
    d0 = degf[0].reshape(n, 1)
    d1 = degf[1].reshape(n, 1)

    br = 2000
    grid = n // br
    row = lambda d: pl.BlockSpec((br,) + d, lambda i: (i,) + (0,) * len(d))
    half = lambda d: pl.BlockSpec((br // 2, d), lambda i: (i, 0))
    full = lambda shp: pl.BlockSpec(shp, lambda i: (0,) * len(shp))

    xs1, dinv = pl.pallas_call(
        _mm_scale_kernel,
        grid=(grid,),
        in_specs=[row((din,)), full((din, dh)), row((1,)), row((1,))],
        out_specs=[row((dh,)), row((1,))],
        out_shape=[
            jax.ShapeDtypeStruct((n, dh), jnp.float32),
            jax.ShapeDtypeStruct((n, 1), jnp.float32),
        ],
    )(x, W1, d0, d1)

    prop1 = _make_prop_kernel(n, dh // 2, ch16, k, 4)
    osum1 = prop1(xs1.reshape(2 * n, dh // 2), idx2, dst16)

    xs2 = pl.pallas_call(
        _combine_mm_kernel,
        grid=(grid,),
        in_specs=[row((dh,)), row((dh,)), row((1,)), full((1, dh)),
                  full((dh, dout))],
        out_specs=row((dout,)),
        out_shape=jax.ShapeDtypeStruct((n, dout), jnp.float32),
    )(osum1, xs1, dinv, b1.reshape(1, dh), W2)

    prop2 = _make_prop_kernel(n, dout // 2, ch16, k, 8)
    osum2 = prop2(xs2.reshape(2 * n, dout // 2), idx2, dst16)

    out = pl.pallas_call(
        _final_kernel,
        grid=(grid,),
        in_specs=[row((dout,)), row((dout,)), row((1,)), full((1, dout))],
        out_specs=row((dout,)),
        out_shape=jax.ShapeDtypeStruct((n, dout), jnp.float32),
    )(osum2, xs2, dinv, b2.reshape(1, dout))

    return out


# speedup vs baseline: 41.3475x; 1.0198x over previous
import functools

import jax
import jax.numpy as jnp
from jax import lax
from jax.experimental import pallas as pl
from jax.experimental.pallas import tpu as pltpu
from jax.experimental.pallas import tpu_sc as plsc

NC = 2
NS = 16
NW = NC * NS

_MESH = plsc.VectorSubcoreMesh(core_axis_name="c", subcore_axis_name="s")


def _make_deg_kernel(n, ch, k):
    npad = ((n + NS * 16 - 1) // (NS * 16)) * (NS * 16)
    rpt = npad // NS
    chh = ch // 2

    @functools.partial(
        pl.kernel,
        out_type=jax.ShapeDtypeStruct((NC * npad,), jnp.float32),
        mesh=_MESH,
        scratch_types=[
            pltpu.VMEM((chh, k), jnp.int32),
            pltpu.VMEM((128,), jnp.float32),
            pltpu.VMEM((rpt,), jnp.float32),
            pltpu.VMEM_SHARED((npad,), jnp.float32),
            pltpu.SemaphoreType.DMA,
        ],
        compiler_params=pltpu.CompilerParams(use_tc_tiling_on_sc=False),
    )
    def deg_kernel(dst_hbm, out_hbm, dst_v, ones_v, stage_v, acc_sh, dsem):
        c = lax.axis_index("c")
        s = lax.axis_index("s")
        wid = c * NS + s

        pltpu.sync_copy(dst_hbm.at[s, pl.ds(c * chh, chh)], dst_v)

        zeros16 = jnp.zeros((16,), jnp.float32)
        ones16 = jnp.ones((16,), jnp.float32)
        for i in range(128 // 16):
            ones_v[pl.ds(i * 16, 16)] = ones16

        def zero_body(i, carry):
            stage_v[pl.ds(i * 16, 16)] = zeros16
            return carry

        lax.fori_loop(0, rpt // 16, zero_body, 0)
        sbase = pl.multiple_of(s * rpt, 8)
        pltpu.sync_copy(stage_v, acc_sh.at[pl.ds(sbase, rpt)])
        plsc.subcore_barrier()

        def edge_body(j, carry):
            pltpu.async_copy(ones_v.at[pl.ds(0, k)], acc_sh.at[dst_v.at[j]],
                             dsem, add=True)
            return carry

        lax.fori_loop(0, chh, edge_body, 0)

        def edge_drain(j, carry):
            pltpu.make_async_copy(ones_v.at[pl.ds(0, k)],
                                  acc_sh.at[dst_v.at[0]], dsem).wait()
            return carry

        lax.fori_loop(0, chh, edge_drain, 0)
        plsc.subcore_barrier()

        pltpu.sync_copy(acc_sh.at[pl.ds(sbase, rpt)], stage_v)
        obase = pl.multiple_of(wid * rpt, 8)
        pltpu.sync_copy(stage_v, out_hbm.at[pl.ds(obase, rpt)])

    return deg_kernel


def _make_prop_kernel(n, d2, ch, k, NB, ow=None):
    ow = ow if ow is not None else 2 * d2
    NL = NB // 2
    rpb = (n // (NS * 8)) * 8
    rem = n - rpb * NS
    sr = 208
    nst = rpb // sr
    assert nst * sr == rpb and rem % 8 == 0 and rem <= sr
    assert ch % NB == 0

    @functools.partial(
        pl.kernel,
        out_type=jax.ShapeDtypeStruct((n, ow), jnp.float32),
        mesh=_MESH,
        scratch_types=[
            pltpu.VMEM((ch, k), jnp.int32),
            pltpu.VMEM((ch, k), jnp.int32),
            [pltpu.VMEM((k, d2), jnp.float32) for _ in range(NB)],
            pltpu.VMEM((sr, d2), jnp.float32),
            pltpu.VMEM_SHARED((n, d2), jnp.float32),
            [pltpu.SemaphoreType.DMA for _ in range(NB)],
            [pltpu.SemaphoreType.DMA for _ in range(NB)],
        ],
        compiler_params=pltpu.CompilerParams(use_tc_tiling_on_sc=False),
    )
    def prop_kernel(t2_hbm, idx_hbm, dst_hbm, out_hbm,
                    idx_v, dst_v, rows, stage_v, acc_sh, gsems, ssems):
        c = lax.axis_index("c")
        s = lax.axis_index("s")

        pltpu.sync_copy(idx_hbm.at[c, s], idx_v)
        pltpu.sync_copy(dst_hbm.at[s], dst_v)

        def tile_chunks(fn):
            for p in range(nst):
                fn(pl.multiple_of(s * rpb + p * sr, 8), sr)
            if rem:
                @pl.when(s == NS - 1)
                def _():
                    fn(NS * rpb, rem)

        zeros16 = jnp.zeros((16,), jnp.float32)

        def zero_body(i, carry):
            for t in range(d2 // 16):
                stage_v[i, pl.ds(t * 16, 16)] = zeros16
            return carry

        lax.fori_loop(0, sr, zero_body, 0)

        def seed0(off, sz):
            pltpu.sync_copy(stage_v.at[pl.ds(0, sz)],
                            acc_sh.at[pl.ds(off, sz)])
        tile_chunks(seed0)
        plsc.subcore_barrier()

        def issue_gather(j, b):
            pltpu.async_copy(t2_hbm.at[idx_v.at[j]], rows[b], gsems[b])

        def wait_gather(b):
            pltpu.make_async_copy(t2_hbm.at[idx_v.at[0]], rows[b],
                                  gsems[b]).wait()

        def issue_scatter(j, b):
            pltpu.async_copy(rows[b], acc_sh.at[dst_v.at[j]], ssems[b],
                             add=True)

        def wait_scatter(b):
            pltpu.make_async_copy(rows[b], acc_sh.at[dst_v.at[0]],
                                  ssems[b]).wait()

        for b0 in range(NL):
            issue_gather(b0, b0)

        def edge_body(jj, carry):
            for b in range(NB):
                j = jj * NB + b
                bp = (b + NL) % NB

                @pl.when(j >= NL)
                def _():
                    wait_scatter(bp)

                @pl.when(j + NL < ch)
                def _():
                    issue_gather(j + NL, bp)

                wait_gather(b)
                issue_scatter(j, b)
            return carry

        lax.fori_loop(0, ch // NB, edge_body, 0)
        for j0 in range(ch - NL, ch):
            wait_scatter(j0 % NB)
        plsc.subcore_barrier()

        def drain(off, sz):
            pltpu.sync_copy(acc_sh.at[pl.ds(off, sz)],
                            stage_v.at[pl.ds(0, sz)])
            pltpu.sync_copy(stage_v.at[pl.ds(0, sz)],
                            out_hbm.at[pl.ds(off, sz), pl.ds(c * d2, d2)])
        tile_chunks(drain)

    return prop_kernel


def _mm_scale_kernel(x_ref, w_ref, d0_ref, d1_ref, xs_ref, dinv_ref):
    h = jnp.dot(x_ref[...], w_ref[...], preferred_element_type=jnp.float32)
    deg = d0_ref[...] + d1_ref[...] + 1.0
    dinv = lax.rsqrt(deg)
    dinv_ref[...] = dinv
    xs_ref[...] = h * dinv


def _combine_mm_kernel(a_ref, sl_ref, dinv_ref, b_ref, w_ref, xs_ref):
    dinv = dinv_ref[...]
    h = jnp.maximum((a_ref[...] + sl_ref[...]) * dinv + b_ref[...], 0.0)
    y = jnp.dot(h, w_ref[...], preferred_element_type=jnp.float32) * dinv
    xs_ref[...] = y


def _final_kernel(a_ref, sl_ref, dinv_ref, b_ref, out_ref):
    o = a_ref[:, : sl_ref.shape[1]] + sl_ref[...]
    o = o * dinv_ref[...] + b_ref[...]
    m = jnp.max(o, axis=1, keepdims=True)
    ex = jnp.exp(o - m)
    lse = jnp.log(jnp.sum(ex, axis=1, keepdims=True)) + m
    out_ref[...] = o - lse


def kernel(x, edge_index, W1, b1, W2, b2):
    n, din = x.shape
    dh = W1.shape[1]
    dout = W2.shape[1]
    e = edge_index.shape[1]

    k = 125
    ch32 = e // (NW * k)
    ch16 = e // (NS * k)
    assert ch32 * NW * k == e

    src = edge_index[0]
    idx2 = jnp.stack([src * 2, src * 2 + 1]).reshape(NC, NS, ch16, k)
    dst16 = edge_index[1].reshape(NS, ch16, k)

    degp = _make_deg_kernel(n, ch16, k)(dst16)
    npad = degp.shape[0] // NC
    degf = degp.reshape(NC, npad)[:, :n]
    d0 = degf[0].reshape(n, 1)
    d1 = degf[1].reshape(n, 1)

    br = 2000
    grid = n // br
    row = lambda d: pl.BlockSpec((br,) + d, lambda i: (i,) + (0,) * len(d))
    half = lambda d: pl.BlockSpec((br // 2, d), lambda i: (i, 0))
    full = lambda shp: pl.BlockSpec(shp, lambda i: (0,) * len(shp))

    xs1, dinv = pl.pallas_call(
        _mm_scale_kernel,
        grid=(grid,),
        in_specs=[row((din,)), full((din, dh)), row((1,)), row((1,))],
        out_specs=[row((dh,)), row((1,))],
        out_shape=[
            jax.ShapeDtypeStruct((n, dh), jnp.float32),
            jax.ShapeDtypeStruct((n, 1), jnp.float32),
        ],
    )(x, W1, d0, d1)

    prop1 = _make_prop_kernel(n, dh // 2, ch16, k, 4)
    osum1 = prop1(xs1.reshape(2 * n, dh // 2), idx2, dst16)

    xs2 = pl.pallas_call(
        _combine_mm_kernel,
        grid=(grid,),
        in_specs=[row((dh,)), row((dh,)), row((1,)), full((1, dh)),
                  full((dh, dout))],
        out_specs=row((dout,)),
        out_shape=jax.ShapeDtypeStruct((n, dout), jnp.float32),
    )(osum1, xs1, dinv, b1.reshape(1, dh), W2)

    prop2 = _make_prop_kernel(n, dout // 2, ch16, k, 8, ow=dh)
    osum2 = prop2(xs2.reshape(2 * n, dout // 2), idx2, dst16)

    out = pl.pallas_call(
        _final_kernel,
        grid=(grid,),
        in_specs=[row((dh,)), row((dout,)), row((1,)), full((1, dout))],
        out_specs=row((dout,)),
        out_shape=jax.ShapeDtypeStruct((n, dout), jnp.float32),
    )(osum2, xs2, dinv, b2.reshape(1, dout))

    return out


# speedup vs baseline: 41.3905x; 1.0010x over previous
import functools

import jax
import jax.numpy as jnp
from jax import lax
from jax.experimental import pallas as pl
from jax.experimental.pallas import tpu as pltpu
from jax.experimental.pallas import tpu_sc as plsc

NC = 2
NS = 16
NW = NC * NS

_MESH = plsc.VectorSubcoreMesh(core_axis_name="c", subcore_axis_name="s")


def _make_deg_kernel(n, ch, k):
    npad = ((n + NS * 16 - 1) // (NS * 16)) * (NS * 16)
    rpt = npad // NS
    chh = ch // 2

    @functools.partial(
        pl.kernel,
        out_type=jax.ShapeDtypeStruct((NC * npad,), jnp.float32),
        mesh=_MESH,
        scratch_types=[
            pltpu.VMEM((chh, k), jnp.int32),
            pltpu.VMEM((128,), jnp.float32),
            pltpu.VMEM((rpt,), jnp.float32),
            pltpu.VMEM_SHARED((npad,), jnp.float32),
            pltpu.SemaphoreType.DMA,
        ],
        compiler_params=pltpu.CompilerParams(use_tc_tiling_on_sc=False),
    )
    def deg_kernel(dst_hbm, out_hbm, dst_v, ones_v, stage_v, acc_sh, dsem):
        c = lax.axis_index("c")
        s = lax.axis_index("s")
        wid = c * NS + s

        pltpu.sync_copy(dst_hbm.at[s, pl.ds(c * chh, chh)], dst_v)

        zeros16 = jnp.zeros((16,), jnp.float32)
        ones16 = jnp.ones((16,), jnp.float32)
        for i in range(128 // 16):
            ones_v[pl.ds(i * 16, 16)] = ones16

        def zero_body(i, carry):
            stage_v[pl.ds(i * 16, 16)] = zeros16
            return carry

        lax.fori_loop(0, rpt // 16, zero_body, 0)
        sbase = pl.multiple_of(s * rpt, 8)
        pltpu.sync_copy(stage_v, acc_sh.at[pl.ds(sbase, rpt)])
        plsc.subcore_barrier()

        def edge_body(j, carry):
            pltpu.async_copy(ones_v.at[pl.ds(0, k)], acc_sh.at[dst_v.at[j]],
                             dsem, add=True)
            return carry

        lax.fori_loop(0, chh, edge_body, 0)

        def edge_drain(j, carry):
            pltpu.make_async_copy(ones_v.at[pl.ds(0, k)],
                                  acc_sh.at[dst_v.at[0]], dsem).wait()
            return carry

        lax.fori_loop(0, chh, edge_drain, 0)
        plsc.subcore_barrier()

        pltpu.sync_copy(acc_sh.at[pl.ds(sbase, rpt)], stage_v)
        obase = pl.multiple_of(wid * rpt, 8)
        pltpu.sync_copy(stage_v, out_hbm.at[pl.ds(obase, rpt)])

    return deg_kernel


def _make_prop_kernel(n, d2, ch, k, NB, ow=None):
    ow = ow if ow is not None else 2 * d2
    NL = NB // 2
    rpb = (n // (NS * 8)) * 8
    rem = n - rpb * NS
    sr = 208
    nst = rpb // sr
    assert nst * sr == rpb and rem % 8 == 0 and rem <= sr
    assert ch % NB == 0

    @functools.partial(
        pl.kernel,
        out_type=jax.ShapeDtypeStruct((n, ow), jnp.float32),
        mesh=_MESH,
        scratch_types=[
            pltpu.VMEM((ch, k), jnp.int32),
            pltpu.VMEM((ch, k), jnp.int32),
            [pltpu.VMEM((k, d2), jnp.float32) for _ in range(NB)],
            pltpu.VMEM((sr, d2), jnp.float32),
            pltpu.VMEM_SHARED((n, d2), jnp.float32),
            [pltpu.SemaphoreType.DMA for _ in range(NB)],
            [pltpu.SemaphoreType.DMA for _ in range(NB)],
        ],
        compiler_params=pltpu.CompilerParams(use_tc_tiling_on_sc=False),
    )
    def prop_kernel(t2_hbm, idx_hbm, dst_hbm, out_hbm,
                    idx_v, dst_v, rows, stage_v, acc_sh, gsems, ssems):
        c = lax.axis_index("c")
        s = lax.axis_index("s")

        pltpu.sync_copy(idx_hbm.at[c, s], idx_v)
        pltpu.sync_copy(dst_hbm.at[s], dst_v)

        def tile_chunks(fn):
            for p in range(nst):
                fn(pl.multiple_of(s * rpb + p * sr, 8), sr)
            if rem:
                @pl.when(s == NS - 1)
                def _():
                    fn(NS * rpb, rem)

        zeros16 = jnp.zeros((16,), jnp.float32)

        def zero_body(i, carry):
            for t in range(d2 // 16):
                stage_v[i, pl.ds(t * 16, 16)] = zeros16
            return carry

        lax.fori_loop(0, sr, zero_body, 0)

        def seed0(off, sz):
            pltpu.sync_copy(stage_v.at[pl.ds(0, sz)],
                            acc_sh.at[pl.ds(off, sz)])
        tile_chunks(seed0)
        plsc.subcore_barrier()

        def issue_gather(j, b):
            pltpu.async_copy(t2_hbm.at[idx_v.at[j]], rows[b], gsems[b])

        def wait_gather(b):
            pltpu.make_async_copy(t2_hbm.at[idx_v.at[0]], rows[b],
                                  gsems[b]).wait()

        def issue_scatter(j, b):
            pltpu.async_copy(rows[b], acc_sh.at[dst_v.at[j]], ssems[b],
                             add=True)

        def wait_scatter(b):
            pltpu.make_async_copy(rows[b], acc_sh.at[dst_v.at[0]],
                                  ssems[b]).wait()

        for b0 in range(NL):
            issue_gather(b0, b0)

        def edge_body(jj, carry):
            for b in range(NB):
                j = jj * NB + b
                bp = (b + NL) % NB

                @pl.when(j >= NL)
                def _():
                    wait_scatter(bp)

                @pl.when(j + NL < ch)
                def _():
                    issue_gather(j + NL, bp)

                wait_gather(b)
                issue_scatter(j, b)
            return carry

        lax.fori_loop(0, ch // NB, edge_body, 0)
        for j0 in range(ch - NL, ch):
            wait_scatter(j0 % NB)
        plsc.subcore_barrier()

        def drain(off, sz):
            pltpu.sync_copy(acc_sh.at[pl.ds(off, sz)],
                            stage_v.at[pl.ds(0, sz)])
            pltpu.sync_copy(stage_v.at[pl.ds(0, sz)],
                            out_hbm.at[pl.ds(off, sz), pl.ds(c * d2, d2)])
        tile_chunks(drain)

    return prop_kernel


def _mm_scale_kernel(x_ref, w_ref, d0_ref, d1_ref, xs_ref, dinv_ref):
    h = jnp.dot(x_ref[...], w_ref[...], preferred_element_type=jnp.float32)
    deg = d0_ref[...] + d1_ref[...] + 1.0
    dinv = lax.rsqrt(deg)
    dinv_ref[...] = dinv
    xs_ref[...] = h * dinv


def _combine_mm_kernel(a_ref, sl_ref, dinv_ref, b_ref, w_ref, xs_ref):
    dinv = dinv_ref[...]
    h = jnp.maximum((a_ref[...] + sl_ref[...]) * dinv + b_ref[...], 0.0)
    y = jnp.dot(h, w_ref[...], preferred_element_type=jnp.float32) * dinv
    xs_ref[...] = y


def _final_kernel(a_ref, sl_ref, dinv_ref, b_ref, out_ref):
    o = a_ref[:, : sl_ref.shape[1]] + sl_ref[...]
    o = o * dinv_ref[...] + b_ref[...]
    m = jnp.max(o, axis=1, keepdims=True)
    ex = jnp.exp(o - m)
    lse = jnp.log(jnp.sum(ex, axis=1, keepdims=True)) + m
    out_ref[...] = o - lse


def kernel(x, edge_index, W1, b1, W2, b2):
    n, din = x.shape
    dh = W1.shape[1]
    dout = W2.shape[1]
    e = edge_index.shape[1]

    k = 125
    ch16 = e // (NS * k)
    assert ch16 * NS * k == e and ch16 % 2 == 0

    src = edge_index[0]
    idx2 = jnp.stack([src * 2, src * 2 + 1]).reshape(NC, NS, ch16, k)
    dst16 = edge_index[1].reshape(NS, ch16, k)

    degp = _make_deg_kernel(n, ch16, k)(dst16)
    npad = degp.shape[0] // NC
    degf = degp.reshape(NC, npad)[:, :n]
    d0 = degf[0].reshape(n, 1)
    d1 = degf[1].reshape(n, 1)

    br = 2000
    grid = n // br
    row = lambda d: pl.BlockSpec((br,) + d, lambda i: (i,) + (0,) * len(d))
    full = lambda shp: pl.BlockSpec(shp, lambda i: (0,) * len(shp))

    xs1, dinv = pl.pallas_call(
        _mm_scale_kernel,
        grid=(grid,),
        in_specs=[row((din,)), full((din, dh)), row((1,)), row((1,))],
        out_specs=[row((dh,)), row((1,))],
        out_shape=[
            jax.ShapeDtypeStruct((n, dh), jnp.float32),
            jax.ShapeDtypeStruct((n, 1), jnp.float32),
        ],
    )(x, W1, d0, d1)

    prop1 = _make_prop_kernel(n, dh // 2, ch16, k, 4)
    osum1 = prop1(xs1.reshape(2 * n, dh // 2), idx2, dst16)

    xs2 = pl.pallas_call(
        _combine_mm_kernel,
        grid=(grid,),
        in_specs=[row((dh,)), row((dh,)), row((1,)), full((1, dh)),
                  full((dh, dout))],
        out_specs=row((dout,)),
        out_shape=jax.ShapeDtypeStruct((n, dout), jnp.float32),
    )(osum1, xs1, dinv, b1.reshape(1, dh), W2)

    prop2 = _make_prop_kernel(n, dout // 2, ch16, k, 8, ow=dh)
    osum2 = prop2(xs2.reshape(2 * n, dout // 2), idx2, dst16)

    out = pl.pallas_call(
        _final_kernel,
        grid=(grid,),
        in_specs=[row((dh,)), row((dout,)), row((1,)), full((1, dout))],
        out_specs=row((dout,)),
        out_shape=jax.ShapeDtypeStruct((n, dout), jnp.float32),
    )(osum2, xs2, dinv, b2.reshape(1, dout))

    return out
